# baseline (speedup 1.0000x reference)
import functools

import jax
import jax.numpy as jnp
from jax import lax
from jax.experimental import pallas as pl
from jax.experimental.pallas import tpu as pltpu
from jax.experimental.pallas import tpu_sc as plsc

VOCAB = 100000
DIM = 4096
BATCH = 4
SEQ = 4096

NUM_CORES = 2
NUM_SUBCORES = 16
NUM_WORKERS = NUM_CORES * NUM_SUBCORES

B_TOTAL = BATCH * SEQ
B_PER_W = B_TOTAL // NUM_WORKERS
CHUNK = 4
NBUF = 7
NCHUNK = B_PER_W // CHUNK
NMAIN = (NCHUNK // NBUF) * NBUF


def _body(x_hbm, table_hbm, out_hbm, idx_v, *scr):
    rows = scr[:NBUF]
    gsem = scr[NBUF:2 * NBUF]
    osem = scr[2 * NBUF:]

    wid = lax.axis_index("s") * NUM_CORES + lax.axis_index("c")
    base = wid * B_PER_W

    pltpu.sync_copy(x_hbm.at[wid], idx_v)

    def gather_start(chunk, b):
        return pltpu.async_copy(table_hbm.at[idx_v.at[chunk]], rows[b], gsem[b])

    def gather_wait(chunk, b):
        pltpu.make_async_copy(table_hbm.at[idx_v.at[chunk]], rows[b], gsem[b]).wait()

    def out_start(chunk, b):
        dst = out_hbm.at[pl.ds(base + chunk * CHUNK, CHUNK)]
        return pltpu.async_copy(rows[b], dst, osem[b])

    def out_wait(chunk, b):
        dst = out_hbm.at[pl.ds(base + chunk * CHUNK, CHUNK)]
        pltpu.make_async_copy(rows[b], dst, osem[b]).wait()

    for b in range(NBUF - 1):
        gather_start(b, b)

    @pl.loop(0, NMAIN, step=NBUF)
    def _(g):
        for b in range(NBUF):
            c = g + b
            gather_wait(c, b)
            out_start(c, b)
            if b == 0:
                @pl.when(c >= 1)
                def _():
                    out_wait(c - 1, (b - 1) % NBUF)
            else:
                out_wait(c - 1, b - 1)

            @pl.when(c + NBUF - 1 < NCHUNK)
            def _():
                gather_start(c + NBUF - 1, (b - 1) % NBUF)

    for c in range(NMAIN, NCHUNK):
        b = c % NBUF
        gather_wait(c, b)
        out_start(c, b)
        out_wait(c - 1, (c - 1) % NBUF)
        if c + NBUF - 1 < NCHUNK:
            gather_start(c + NBUF - 1, (c - 1) % NBUF)
    out_wait(NCHUNK - 1, (NCHUNK - 1) % NBUF)


@jax.jit
def _lookup(x_flat, table):
    x_flat = x_flat.reshape(NUM_WORKERS, NCHUNK, CHUNK)
    mesh = plsc.VectorSubcoreMesh(
        core_axis_name="c", subcore_axis_name="s",
        num_cores=NUM_CORES, num_subcores=NUM_SUBCORES)
    fn = pl.kernel(
        _body,
        out_type=jax.ShapeDtypeStruct((B_TOTAL, DIM), jnp.float32),
        mesh=mesh,
        scratch_types=[
            pltpu.VMEM((NCHUNK, CHUNK), jnp.int32),
        ] + [pltpu.VMEM((CHUNK, DIM), jnp.float32)] * NBUF
          + [pltpu.SemaphoreType.DMA] * (2 * NBUF),
    )
    return fn(x_flat, table)


def kernel(x, table):
    x_flat = x.reshape(-1).astype(jnp.int32)
    out = _lookup(x_flat, table)
    return out.reshape(BATCH, SEQ, DIM)


# speedup vs baseline: 1.0028x; 1.0028x over previous
import functools

import jax
import jax.numpy as jnp
from jax import lax
from jax.experimental import pallas as pl
from jax.experimental.pallas import tpu as pltpu
from jax.experimental.pallas import tpu_sc as plsc

VOCAB = 100000
DIM = 4096
BATCH = 4
SEQ = 4096

NUM_CORES = 2
NUM_SUBCORES = 16
NUM_WORKERS = NUM_CORES * NUM_SUBCORES

B_TOTAL = BATCH * SEQ
B_PER_W = B_TOTAL // NUM_WORKERS
CHUNK = 8
NBUF = 3
NCHUNK = B_PER_W // CHUNK
NMAIN = (NCHUNK // NBUF) * NBUF


def _body(x_hbm, table_hbm, out_hbm, idx_v, *scr):
    rows = scr[:NBUF]
    gsem = scr[NBUF:2 * NBUF]
    osem = scr[2 * NBUF:]

    wid = lax.axis_index("s") * NUM_CORES + lax.axis_index("c")
    base = wid * B_PER_W

    pltpu.sync_copy(x_hbm.at[wid], idx_v)

    def gather_start(chunk, b):
        return pltpu.async_copy(table_hbm.at[idx_v.at[chunk]], rows[b], gsem[b])

    def gather_wait(chunk, b):
        pltpu.make_async_copy(table_hbm.at[idx_v.at[chunk]], rows[b], gsem[b]).wait()

    def out_start(chunk, b):
        dst = out_hbm.at[pl.ds(base + chunk * CHUNK, CHUNK)]
        return pltpu.async_copy(rows[b], dst, osem[b])

    def out_wait(chunk, b):
        dst = out_hbm.at[pl.ds(base + chunk * CHUNK, CHUNK)]
        pltpu.make_async_copy(rows[b], dst, osem[b]).wait()

    for b in range(NBUF - 1):
        gather_start(b, b)

    @pl.loop(0, NMAIN, step=NBUF)
    def _(g):
        for b in range(NBUF):
            c = g + b
            gather_wait(c, b)
            out_start(c, b)
            if b == 0:
                @pl.when(c >= 1)
                def _():
                    out_wait(c - 1, (b - 1) % NBUF)
            else:
                out_wait(c - 1, b - 1)

            @pl.when(c + NBUF - 1 < NCHUNK)
            def _():
                gather_start(c + NBUF - 1, (b - 1) % NBUF)

    for c in range(NMAIN, NCHUNK):
        b = c % NBUF
        gather_wait(c, b)
        out_start(c, b)
        out_wait(c - 1, (c - 1) % NBUF)
        if c + NBUF - 1 < NCHUNK:
            gather_start(c + NBUF - 1, (c - 1) % NBUF)
    out_wait(NCHUNK - 1, (NCHUNK - 1) % NBUF)


@jax.jit
def _lookup(x_flat, table):
    x_flat = x_flat.reshape(NUM_WORKERS, NCHUNK, CHUNK)
    mesh = plsc.VectorSubcoreMesh(
        core_axis_name="c", subcore_axis_name="s",
        num_cores=NUM_CORES, num_subcores=NUM_SUBCORES)
    fn = pl.kernel(
        _body,
        out_type=jax.ShapeDtypeStruct((B_TOTAL, DIM), jnp.float32),
        mesh=mesh,
        scratch_types=[
            pltpu.VMEM((NCHUNK, CHUNK), jnp.int32),
        ] + [pltpu.VMEM((CHUNK, DIM), jnp.float32)] * NBUF
          + [pltpu.SemaphoreType.DMA] * (2 * NBUF),
    )
    return fn(x_flat, table)


def kernel(x, table):
    x_flat = x.reshape(-1).astype(jnp.int32)
    out = _lookup(x_flat, table)
    return out.reshape(BATCH, SEQ, DIM)


# speedup vs baseline: 1.0088x; 1.0060x over previous
import jax
import jax.numpy as jnp
from jax import lax
from jax.experimental import pallas as pl
from jax.experimental.pallas import tpu as pltpu
from jax.experimental.pallas import tpu_sc as plsc

VOCAB = 100000
DIM = 4096
BATCH = 4
SEQ = 4096

NUM_CORES = 2
NUM_SUBCORES = 16
NUM_WORKERS = NUM_CORES * NUM_SUBCORES

B_TOTAL = BATCH * SEQ
B_PER_W = B_TOTAL // NUM_WORKERS
CHUNK = 8
NBUF = 2
NCHUNK = B_PER_W // CHUNK


def _body(x_hbm, table_hbm, out_hbm, idx_v, *scr):
    rows = scr[:NBUF]
    gsem = scr[NBUF:2 * NBUF]
    osem = scr[2 * NBUF:]

    wid = lax.axis_index("s") * NUM_CORES + lax.axis_index("c")
    base = wid * B_PER_W

    pltpu.sync_copy(x_hbm.at[pl.ds(base, B_PER_W)], idx_v)

    def gather_start(chunk, b):
        idx_slice = idx_v.at[pl.ds(chunk * CHUNK, CHUNK)]
        return pltpu.async_copy(table_hbm.at[idx_slice], rows[b], gsem[b])

    def gather_wait(chunk, b):
        idx_slice = idx_v.at[pl.ds(chunk * CHUNK, CHUNK)]
        pltpu.make_async_copy(table_hbm.at[idx_slice], rows[b], gsem[b]).wait()

    def out_start(chunk, b):
        dst = out_hbm.at[pl.ds(base + chunk * CHUNK, CHUNK)]
        return pltpu.async_copy(rows[b], dst, osem[b])

    def out_wait(chunk, b):
        dst = out_hbm.at[pl.ds(base + chunk * CHUNK, CHUNK)]
        pltpu.make_async_copy(rows[b], dst, osem[b]).wait()

    for b in range(NBUF):
        gather_start(b, b)

    @pl.loop(0, NCHUNK - NBUF, step=NBUF)
    def _(g):
        for b in range(NBUF):
            c = g + b
            gather_wait(c, b)
            out_start(c, b)
            out_wait(c, b)
            gather_start(c + NBUF, b)

    for b in range(NBUF):
        c = NCHUNK - NBUF + b
        gather_wait(c, b)
        out_start(c, b)
        out_wait(c, b)


@jax.jit
def _lookup(x_flat, table):
    mesh = plsc.VectorSubcoreMesh(
        core_axis_name="c", subcore_axis_name="s",
        num_cores=NUM_CORES, num_subcores=NUM_SUBCORES)
    fn = pl.kernel(
        _body,
        out_type=jax.ShapeDtypeStruct((B_TOTAL, DIM), jnp.float32),
        mesh=mesh,
        scratch_types=[
            pltpu.VMEM((B_PER_W,), jnp.int32),
        ] + [pltpu.VMEM((CHUNK, DIM), jnp.float32)] * NBUF
          + [pltpu.SemaphoreType.DMA] * (2 * NBUF),
    )
    return fn(x_flat, table)


def kernel(x, table):
    x_flat = x.reshape(-1).astype(jnp.int32)
    out = _lookup(x_flat, table)
    return out.reshape(BATCH, SEQ, DIM)
